# baseline (speedup 1.0000x reference)
import functools

import jax
import jax.numpy as jnp
from jax import lax
from jax.experimental import pallas as pl
from jax.experimental.pallas import tpu as pltpu
from jax.experimental.pallas import tpu_sc as plsc

N = 10000
NP = 10240
D = 128
DFF = 512
E = 320000
EB = 128
KB = 80
NW = 32
EP = NW * KB * EB
RPT = NP // 16
BLK = 1000
EPS = 1e-5

_mesh = plsc.VectorSubcoreMesh(core_axis_name="c", subcore_axis_name="s")


@functools.partial(
    pl.kernel,
    mesh=_mesh,
    out_type=jax.ShapeDtypeStruct((2, NP), jnp.float32),
    scratch_types=[
        pltpu.VMEM((KB, EB), jnp.int32),
        pltpu.VMEM((EB,), jnp.float32),
        pltpu.VMEM((RPT,), jnp.float32),
        pltpu.SemaphoreType.DMA,
        pltpu.VMEM_SHARED((NP,), jnp.float32),
    ],
)
def _deg_kernel(dst_hbm, out_hbm, idx_v, ones_v, stage_v, dsem, deg_sh):
    c = lax.axis_index("c")
    s = lax.axis_index("s")
    wid = s * 2 + c

    def _ones(i, _):
        ones_v[pl.ds(i * 16, 16)] = jnp.ones((16,), jnp.float32)
        return 0

    lax.fori_loop(0, EB // 16, _ones, 0)

    def _z(k, _):
        stage_v[pl.ds(k * 16, 16)] = jnp.zeros((16,), jnp.float32)
        return 0

    lax.fori_loop(0, RPT // 16, _z, 0)

    pltpu.sync_copy(stage_v, deg_sh.at[pl.ds(s * RPT, RPT)])
    plsc.subcore_barrier()

    pltpu.sync_copy(dst_hbm.at[pl.ds(wid * KB, KB)], idx_v)

    def _scat(g, _):
        for b in range(16):
            pltpu.async_copy(ones_v, deg_sh.at[idx_v.at[g * 16 + b]], dsem,
                             add=True)
        for b in range(16):
            pltpu.make_async_copy(ones_v, deg_sh.at[idx_v.at[0]],
                                  dsem).wait()
        return 0

    lax.fori_loop(0, KB // 16, _scat, 0)
    plsc.subcore_barrier()

    pltpu.sync_copy(deg_sh.at[pl.ds(s * RPT, RPT)],
                    out_hbm.at[c, pl.ds(s * RPT, RPT)])


@functools.partial(
    pl.kernel,
    mesh=_mesh,
    out_type=jax.ShapeDtypeStruct((2, NP, D), jnp.float32),
    scratch_types=[
        pltpu.VMEM((KB // 2, EB), jnp.int32),
        pltpu.VMEM((KB // 2, EB), jnp.int32),
        pltpu.VMEM((EB, D), jnp.float32),
        pltpu.VMEM((EB, D), jnp.float32),
        pltpu.SemaphoreType.DMA,
        pltpu.SemaphoreType.DMA,
        pltpu.VMEM_SHARED((NP, D), jnp.float32),
    ],
)
def _edge_kernel(h2r_hbm, src_hbm, dst_hbm, out_hbm, src_v, dst_v, rows_v,
                 rows_w, sema, semb, acc_sh):
    c = lax.axis_index("c")
    s = lax.axis_index("s")
    wid = s * 2 + c
    tbl = h2r_hbm.at[wid % 4]

    def _zrow(r, _):
        def _z(k, _2):
            rows_v[r, pl.ds(k * 16, 16)] = jnp.zeros((16,), jnp.float32)
            return 0

        lax.fori_loop(0, D // 16, _z, 0)
        return 0

    lax.fori_loop(0, EB, _zrow, 0)

    def _zacc(k, _):
        pltpu.sync_copy(rows_v, acc_sh.at[pl.ds(s * RPT + k * EB, EB)])
        return 0

    lax.fori_loop(0, RPT // EB, _zacc, 0)
    plsc.subcore_barrier()

    HKB = KB // 2
    for p in range(2):
        pltpu.sync_copy(src_hbm.at[pl.ds(wid * KB + p * HKB, HKB)], src_v)
        pltpu.sync_copy(dst_hbm.at[pl.ds(wid * KB + p * HKB, HKB)], dst_v)
        pltpu.async_copy(tbl.at[src_v.at[0]], rows_v, sema)

        def _edge(j, _):
            ja = 2 * j
            jb = ja + 1
            jc = jnp.minimum(ja + 2, HKB - 1)
            pltpu.make_async_copy(tbl.at[src_v.at[ja]], rows_v,
                                  sema).wait()
            pltpu.async_copy(tbl.at[src_v.at[jb]], rows_w, semb)
            pltpu.sync_copy(rows_v, acc_sh.at[dst_v.at[ja]], add=True)
            pltpu.make_async_copy(tbl.at[src_v.at[jb]], rows_w,
                                  semb).wait()
            pltpu.async_copy(tbl.at[src_v.at[jc]], rows_v, sema)
            pltpu.sync_copy(rows_w, acc_sh.at[dst_v.at[jb]], add=True)
            return 0

        lax.fori_loop(0, HKB // 2, _edge, 0)
        pltpu.make_async_copy(tbl.at[src_v.at[HKB - 1]], rows_v,
                              sema).wait()
    plsc.subcore_barrier()

    pltpu.sync_copy(acc_sh.at[pl.ds(s * RPT, RPT)],
                    out_hbm.at[c, pl.ds(s * RPT, RPT)])


def _prep_body(x_ref, w_ref, degp_ref, h2r_ref, dinv_ref):
    deg = degp_ref[0] + degp_ref[1] + 1.0
    dinv = lax.rsqrt(deg)
    dinv_ref[...] = dinv
    h = jnp.dot(x_ref[...], w_ref[...], preferred_element_type=jnp.float32)
    h2 = h * dinv
    for k in range(4):
        h2r_ref[k] = h2


def _prep(x, w, degp):
    return pl.pallas_call(
        _prep_body,
        grid=(N // BLK,),
        in_specs=[
            pl.BlockSpec((BLK, D), lambda i: (i, 0)),
            pl.BlockSpec((D, D), lambda i: (0, 0)),
            pl.BlockSpec((2, BLK, 1), lambda i: (0, i, 0)),
        ],
        out_specs=[
            pl.BlockSpec((4, BLK, D), lambda i: (0, i, 0)),
            pl.BlockSpec((BLK, 1), lambda i: (i, 0)),
        ],
        out_shape=[
            jax.ShapeDtypeStruct((4, N, D), jnp.float32),
            jax.ShapeDtypeStruct((N, 1), jnp.float32),
        ],
    )(x, w, degp)


def _mega_body(x_ref, parts_ref, h2_ref, dinv_ref, bg_ref, w1_ref, b1_ref,
               w2_ref, b2_ref, bnw_ref, bnb_ref, o_ref, y_s, z_s, st_s,
               st2_s):
    p = pl.program_id(0)
    i = pl.program_id(1)
    rows = pl.ds(i * BLK, BLK)

    @pl.when(jnp.logical_and(p == 0, i == 0))
    def _():
        st_s[...] = jnp.zeros_like(st_s)
        st2_s[...] = jnp.zeros_like(st2_s)

    @pl.when(p == 0)
    def _():
        gcn = dinv_ref[...] * (parts_ref[0] + parts_ref[1] + h2_ref[0])
        y = x_ref[...] + gcn + bg_ref[0:1, :]
        y_s[rows, :] = y
        st_s[0:1, :] += jnp.sum(y, axis=0, keepdims=True)
        st_s[1:2, :] += jnp.sum(y * y, axis=0, keepdims=True)

    @pl.when(p == 1)
    def _():
        mean = st_s[0:1, :] / N
        var = st_s[1:2, :] / N - mean * mean
        a = bnw_ref[0:1, :] * lax.rsqrt(var + EPS)
        cshift = bnb_ref[0:1, :] - mean * a
        h1 = y_s[rows, :] * a + cshift
        f1 = lax.dot_general(h1, w1_ref[...], (((1,), (1,)), ((), ())),
                             preferred_element_type=jnp.float32)
        f1 = jnp.maximum(f1 + b1_ref[0:1, :], 0.0)
        z = h1 + lax.dot_general(f1, w2_ref[...], (((1,), (1,)), ((), ())),
                                 preferred_element_type=jnp.float32)
        z = z + b2_ref[0:1, :]
        z_s[rows, :] = z
        st2_s[0:1, :] += jnp.sum(z, axis=0, keepdims=True)
        st2_s[1:2, :] += jnp.sum(z * z, axis=0, keepdims=True)

    @pl.when(p == 2)
    def _():
        mean = st2_s[0:1, :] / N
        var = st2_s[1:2, :] / N - mean * mean
        a = bnw_ref[0:1, :] * lax.rsqrt(var + EPS)
        cshift = bnb_ref[0:1, :] - mean * a
        o_ref[...] = z_s[rows, :] * a + cshift


def _mega(x, parts, h2, dinv, bg8, w1, b1_8, w2, b2_8, bnw8, bnb8):
    p0 = lambda p, i: (jnp.where(p == 0, i, 0), 0)
    return pl.pallas_call(
        _mega_body,
        grid=(3, N // BLK),
        in_specs=[
            pl.BlockSpec((BLK, D), p0),
            pl.BlockSpec((2, BLK, D),
                         lambda p, i: (0, jnp.where(p == 0, i, 0), 0)),
            pl.BlockSpec((1, BLK, D),
                         lambda p, i: (0, jnp.where(p == 0, i, 0), 0)),
            pl.BlockSpec((BLK, 1), p0),
            pl.BlockSpec((8, D), lambda p, i: (0, 0)),
            pl.BlockSpec((DFF, D), lambda p, i: (0, 0)),
            pl.BlockSpec((8, DFF), lambda p, i: (0, 0)),
            pl.BlockSpec((D, DFF), lambda p, i: (0, 0)),
            pl.BlockSpec((8, D), lambda p, i: (0, 0)),
            pl.BlockSpec((8, D), lambda p, i: (0, 0)),
            pl.BlockSpec((8, D), lambda p, i: (0, 0)),
        ],
        out_specs=pl.BlockSpec((BLK, D), lambda p, i: (jnp.where(p == 2, i, 0), 0)),
        out_shape=jax.ShapeDtypeStruct((N, D), jnp.float32),
        scratch_shapes=[
            pltpu.VMEM((N, D), jnp.float32),
            pltpu.VMEM((N, D), jnp.float32),
            pltpu.VMEM((8, 128), jnp.float32),
            pltpu.VMEM((8, 128), jnp.float32),
        ],
    )(x, parts, h2, dinv, bg8, w1, b1_8, w2, b2_8, bnw8, bnb8)


def _comb_body(x_ref, parts_ref, h2_ref, dinv_ref, bg_ref, y_ref, st_ref):
    i = pl.program_id(0)

    @pl.when(i == 0)
    def _():
        st_ref[...] = jnp.zeros_like(st_ref)

    gcn = dinv_ref[...] * (parts_ref[0] + parts_ref[1] + h2_ref[...])
    y = x_ref[...] + gcn + bg_ref[0:1, :]
    y_ref[...] = y
    st_ref[0:1, :] += jnp.sum(y, axis=0, keepdims=True)
    st_ref[1:2, :] += jnp.sum(y * y, axis=0, keepdims=True)


def _combine(x, parts, h2, dinv, bg8):
    return pl.pallas_call(
        _comb_body,
        grid=(N // BLK,),
        in_specs=[
            pl.BlockSpec((BLK, D), lambda i: (i, 0)),
            pl.BlockSpec((2, BLK, D), lambda i: (0, i, 0)),
            pl.BlockSpec((BLK, D), lambda i: (i, 0)),
            pl.BlockSpec((BLK, 1), lambda i: (i, 0)),
            pl.BlockSpec((8, 128), lambda i: (0, 0)),
        ],
        out_specs=[
            pl.BlockSpec((BLK, D), lambda i: (i, 0)),
            pl.BlockSpec((8, 128), lambda i: (0, 0)),
        ],
        out_shape=[
            jax.ShapeDtypeStruct((N, D), jnp.float32),
            jax.ShapeDtypeStruct((8, 128), jnp.float32),
        ],
    )(x, parts, h2, dinv, bg8)


def _ff_body(y_ref, st_ref, w1_ref, b1_ref, w2_ref, b2_ref, bnw_ref, bnb_ref,
             z_ref, st2_ref):
    i = pl.program_id(0)

    @pl.when(i == 0)
    def _():
        st2_ref[...] = jnp.zeros_like(st2_ref)

    mean = st_ref[0:1, :] / N
    var = st_ref[1:2, :] / N - mean * mean
    a = bnw_ref[0:1, :] * lax.rsqrt(var + EPS)
    cshift = bnb_ref[0:1, :] - mean * a
    h1 = y_ref[...] * a + cshift
    f1 = lax.dot_general(h1, w1_ref[...], (((1,), (1,)), ((), ())),
                         preferred_element_type=jnp.float32)
    f1 = jnp.maximum(f1 + b1_ref[0:1, :], 0.0)
    z = h1 + lax.dot_general(f1, w2_ref[...], (((1,), (1,)), ((), ())),
                             preferred_element_type=jnp.float32)
    z = z + b2_ref[0:1, :]
    z_ref[...] = z
    st2_ref[0:1, :] += jnp.sum(z, axis=0, keepdims=True)
    st2_ref[1:2, :] += jnp.sum(z * z, axis=0, keepdims=True)


def _ffn(y, st, w1, b1_8, w2, b2_8, bnw8, bnb8):
    return pl.pallas_call(
        _ff_body,
        grid=(N // BLK,),
        in_specs=[
            pl.BlockSpec((BLK, D), lambda i: (i, 0)),
            pl.BlockSpec((8, 128), lambda i: (0, 0)),
            pl.BlockSpec((DFF, D), lambda i: (0, 0)),
            pl.BlockSpec((8, DFF), lambda i: (0, 0)),
            pl.BlockSpec((D, DFF), lambda i: (0, 0)),
            pl.BlockSpec((8, 128), lambda i: (0, 0)),
            pl.BlockSpec((8, 128), lambda i: (0, 0)),
            pl.BlockSpec((8, 128), lambda i: (0, 0)),
        ],
        out_specs=[
            pl.BlockSpec((BLK, D), lambda i: (i, 0)),
            pl.BlockSpec((8, 128), lambda i: (0, 0)),
        ],
        out_shape=[
            jax.ShapeDtypeStruct((N, D), jnp.float32),
            jax.ShapeDtypeStruct((8, 128), jnp.float32),
        ],
    )(y, st, w1, b1_8, w2, b2_8, bnw8, bnb8)


def _bn2_body(z_ref, st2_ref, bnw_ref, bnb_ref, o_ref):
    mean = st2_ref[0:1, :] / N
    var = st2_ref[1:2, :] / N - mean * mean
    a = bnw_ref[0:1, :] * lax.rsqrt(var + EPS)
    cshift = bnb_ref[0:1, :] - mean * a
    o_ref[...] = z_ref[...] * a + cshift


def _bn2(z, st2, bnw8, bnb8):
    return pl.pallas_call(
        _bn2_body,
        grid=(N // BLK,),
        in_specs=[
            pl.BlockSpec((BLK, D), lambda i: (i, 0)),
            pl.BlockSpec((8, 128), lambda i: (0, 0)),
            pl.BlockSpec((8, 128), lambda i: (0, 0)),
            pl.BlockSpec((8, 128), lambda i: (0, 0)),
        ],
        out_specs=pl.BlockSpec((BLK, D), lambda i: (i, 0)),
        out_shape=jax.ShapeDtypeStruct((N, D), jnp.float32),
    )(z, st2, bnw8, bnb8)


def kernel(x, edge_index, W_gcn, b_gcn, bn_w, bn_b, W1, b1, W2, b2):
    src = edge_index[0].astype(jnp.int32)
    dst = edge_index[1].astype(jnp.int32)
    pidx = jnp.arange(EP - E, dtype=jnp.int32) % 128
    srcp = jnp.concatenate([src, pidx]).reshape(NW * KB, EB)
    dstp = jnp.concatenate([dst, N + pidx]).reshape(NW * KB, EB)

    degp = _deg_kernel(dstp).reshape(2, NP, 1)
    h2, dinv = _prep(x, W_gcn, degp)
    parts = _edge_kernel(h2, srcp, dstp)

    bg8 = jnp.broadcast_to(b_gcn.reshape(1, D), (8, D))
    bnw8 = jnp.broadcast_to(bn_w.reshape(1, D), (8, D))
    bnb8 = jnp.broadcast_to(bn_b.reshape(1, D), (8, D))
    b1_8 = jnp.broadcast_to(b1.reshape(1, DFF), (8, DFF))
    b2_8 = jnp.broadcast_to(b2.reshape(1, D), (8, D))

    return _mega(x, parts, h2, dinv, bg8, W1, b1_8, W2, b2_8, bnw8, bnb8)


# speedup vs baseline: 1.0194x; 1.0194x over previous
import functools

import jax
import jax.numpy as jnp
from jax import lax
from jax.experimental import pallas as pl
from jax.experimental.pallas import tpu as pltpu
from jax.experimental.pallas import tpu_sc as plsc

N = 10000
NP = 10240
D = 128
DFF = 512
E = 320000
EB = 128
KB = 80
NW = 32
EP = NW * KB * EB
RPT = NP // 16
BLK = 1000
EPS = 1e-5

_mesh = plsc.VectorSubcoreMesh(core_axis_name="c", subcore_axis_name="s")


@functools.partial(
    pl.kernel,
    mesh=_mesh,
    out_type=jax.ShapeDtypeStruct((2, NP), jnp.float32),
    scratch_types=[
        pltpu.VMEM((KB, EB), jnp.int32),
        pltpu.VMEM((EB,), jnp.float32),
        pltpu.VMEM((RPT,), jnp.float32),
        pltpu.SemaphoreType.DMA,
        pltpu.VMEM_SHARED((NP,), jnp.float32),
    ],
)
def _deg_kernel(dst_hbm, out_hbm, idx_v, ones_v, stage_v, dsem, deg_sh):
    c = lax.axis_index("c")
    s = lax.axis_index("s")
    wid = s * 2 + c

    def _ones(i, _):
        ones_v[pl.ds(i * 16, 16)] = jnp.ones((16,), jnp.float32)
        return 0

    lax.fori_loop(0, EB // 16, _ones, 0)

    def _z(k, _):
        stage_v[pl.ds(k * 16, 16)] = jnp.zeros((16,), jnp.float32)
        return 0

    lax.fori_loop(0, RPT // 16, _z, 0)

    pltpu.sync_copy(stage_v, deg_sh.at[pl.ds(s * RPT, RPT)])
    plsc.subcore_barrier()

    pltpu.sync_copy(dst_hbm.at[pl.ds(wid * KB, KB)], idx_v)

    def _scat(g, _):
        for b in range(16):
            pltpu.async_copy(ones_v, deg_sh.at[idx_v.at[g * 16 + b]], dsem,
                             add=True)
        for b in range(16):
            pltpu.make_async_copy(ones_v, deg_sh.at[idx_v.at[0]],
                                  dsem).wait()
        return 0

    lax.fori_loop(0, KB // 16, _scat, 0)
    plsc.subcore_barrier()

    pltpu.sync_copy(deg_sh.at[pl.ds(s * RPT, RPT)],
                    out_hbm.at[c, pl.ds(s * RPT, RPT)])


@functools.partial(
    pl.kernel,
    mesh=_mesh,
    out_type=jax.ShapeDtypeStruct((2, NP, D), jnp.float32),
    scratch_types=[
        pltpu.VMEM((KB // 2, EB), jnp.int32),
        pltpu.VMEM((KB // 2, EB), jnp.int32),
        pltpu.VMEM((EB, D), jnp.float32),
        pltpu.VMEM((EB, D), jnp.float32),
        pltpu.SemaphoreType.DMA,
        pltpu.SemaphoreType.DMA,
        pltpu.VMEM_SHARED((NP, D), jnp.float32),
    ],
)
def _edge_kernel(tbl, src_hbm, dst_hbm, out_hbm, src_v, dst_v, rows_v,
                 rows_w, sema, semb, acc_sh):
    c = lax.axis_index("c")
    s = lax.axis_index("s")
    wid = s * 2 + c

    def _zrow(r, _):
        def _z(k, _2):
            rows_v[r, pl.ds(k * 16, 16)] = jnp.zeros((16,), jnp.float32)
            return 0

        lax.fori_loop(0, D // 16, _z, 0)
        return 0

    lax.fori_loop(0, EB, _zrow, 0)

    def _zacc(k, _):
        pltpu.sync_copy(rows_v, acc_sh.at[pl.ds(s * RPT + k * EB, EB)])
        return 0

    lax.fori_loop(0, RPT // EB, _zacc, 0)
    plsc.subcore_barrier()

    HKB = KB // 2
    for p in range(2):
        pltpu.sync_copy(src_hbm.at[pl.ds(wid * KB + p * HKB, HKB)], src_v)
        pltpu.sync_copy(dst_hbm.at[pl.ds(wid * KB + p * HKB, HKB)], dst_v)
        pltpu.async_copy(tbl.at[src_v.at[0]], rows_v, sema)

        def _edge(j, _):
            ja = 2 * j
            jb = ja + 1
            jc = jnp.minimum(ja + 2, HKB - 1)
            pltpu.make_async_copy(tbl.at[src_v.at[ja]], rows_v,
                                  sema).wait()
            pltpu.async_copy(tbl.at[src_v.at[jb]], rows_w, semb)
            pltpu.sync_copy(rows_v, acc_sh.at[dst_v.at[ja]], add=True)
            pltpu.make_async_copy(tbl.at[src_v.at[jb]], rows_w,
                                  semb).wait()
            pltpu.async_copy(tbl.at[src_v.at[jc]], rows_v, sema)
            pltpu.sync_copy(rows_w, acc_sh.at[dst_v.at[jb]], add=True)
            return 0

        lax.fori_loop(0, HKB // 2, _edge, 0)
        pltpu.make_async_copy(tbl.at[src_v.at[HKB - 1]], rows_v,
                              sema).wait()
    plsc.subcore_barrier()

    pltpu.sync_copy(acc_sh.at[pl.ds(s * RPT, RPT)],
                    out_hbm.at[c, pl.ds(s * RPT, RPT)])


def _prep_body(x_ref, w_ref, degp_ref, h2r_ref, dinv_ref):
    deg = degp_ref[0] + degp_ref[1] + 1.0
    dinv = lax.rsqrt(deg)
    dinv_ref[...] = dinv
    h = jnp.dot(x_ref[...], w_ref[...], preferred_element_type=jnp.float32)
    h2r_ref[...] = h * dinv


def _prep(x, w, degp):
    return pl.pallas_call(
        _prep_body,
        grid=(N // BLK,),
        in_specs=[
            pl.BlockSpec((BLK, D), lambda i: (i, 0)),
            pl.BlockSpec((D, D), lambda i: (0, 0)),
            pl.BlockSpec((2, BLK, 1), lambda i: (0, i, 0)),
        ],
        out_specs=[
            pl.BlockSpec((BLK, D), lambda i: (i, 0)),
            pl.BlockSpec((BLK, 1), lambda i: (i, 0)),
        ],
        out_shape=[
            jax.ShapeDtypeStruct((N, D), jnp.float32),
            jax.ShapeDtypeStruct((N, 1), jnp.float32),
        ],
    )(x, w, degp)


def _mega_body(x_ref, parts_ref, h2_ref, dinv_ref, bg_ref, w1_ref, b1_ref,
               w2_ref, b2_ref, bnw_ref, bnb_ref, o_ref, y_s, z_s, st_s,
               st2_s):
    p = pl.program_id(0)
    i = pl.program_id(1)
    rows = pl.ds(i * BLK, BLK)

    @pl.when(jnp.logical_and(p == 0, i == 0))
    def _():
        st_s[...] = jnp.zeros_like(st_s)
        st2_s[...] = jnp.zeros_like(st2_s)

    @pl.when(p == 0)
    def _():
        gcn = dinv_ref[...] * (parts_ref[0] + parts_ref[1] + h2_ref[...])
        y = x_ref[...] + gcn + bg_ref[0:1, :]
        y_s[rows, :] = y
        st_s[0:1, :] += jnp.sum(y, axis=0, keepdims=True)
        st_s[1:2, :] += jnp.sum(y * y, axis=0, keepdims=True)

    @pl.when(p == 1)
    def _():
        mean = st_s[0:1, :] / N
        var = st_s[1:2, :] / N - mean * mean
        a = bnw_ref[0:1, :] * lax.rsqrt(var + EPS)
        cshift = bnb_ref[0:1, :] - mean * a
        h1 = y_s[rows, :] * a + cshift
        f1 = lax.dot_general(h1, w1_ref[...], (((1,), (1,)), ((), ())),
                             preferred_element_type=jnp.float32)
        f1 = jnp.maximum(f1 + b1_ref[0:1, :], 0.0)
        z = h1 + lax.dot_general(f1, w2_ref[...], (((1,), (1,)), ((), ())),
                                 preferred_element_type=jnp.float32)
        z = z + b2_ref[0:1, :]
        z_s[rows, :] = z
        st2_s[0:1, :] += jnp.sum(z, axis=0, keepdims=True)
        st2_s[1:2, :] += jnp.sum(z * z, axis=0, keepdims=True)

    @pl.when(p == 2)
    def _():
        mean = st2_s[0:1, :] / N
        var = st2_s[1:2, :] / N - mean * mean
        a = bnw_ref[0:1, :] * lax.rsqrt(var + EPS)
        cshift = bnb_ref[0:1, :] - mean * a
        o_ref[...] = z_s[rows, :] * a + cshift


def _mega(x, parts, h2, dinv, bg8, w1, b1_8, w2, b2_8, bnw8, bnb8):
    p0 = lambda p, i: (jnp.where(p == 0, i, 0), 0)
    return pl.pallas_call(
        _mega_body,
        grid=(3, N // BLK),
        in_specs=[
            pl.BlockSpec((BLK, D), p0),
            pl.BlockSpec((2, BLK, D),
                         lambda p, i: (0, jnp.where(p == 0, i, 0), 0)),
            pl.BlockSpec((BLK, D), p0),
            pl.BlockSpec((BLK, 1), p0),
            pl.BlockSpec((8, D), lambda p, i: (0, 0)),
            pl.BlockSpec((DFF, D), lambda p, i: (0, 0)),
            pl.BlockSpec((8, DFF), lambda p, i: (0, 0)),
            pl.BlockSpec((D, DFF), lambda p, i: (0, 0)),
            pl.BlockSpec((8, D), lambda p, i: (0, 0)),
            pl.BlockSpec((8, D), lambda p, i: (0, 0)),
            pl.BlockSpec((8, D), lambda p, i: (0, 0)),
        ],
        out_specs=pl.BlockSpec((BLK, D), lambda p, i: (jnp.where(p == 2, i, 0), 0)),
        out_shape=jax.ShapeDtypeStruct((N, D), jnp.float32),
        scratch_shapes=[
            pltpu.VMEM((N, D), jnp.float32),
            pltpu.VMEM((N, D), jnp.float32),
            pltpu.VMEM((8, 128), jnp.float32),
            pltpu.VMEM((8, 128), jnp.float32),
        ],
    )(x, parts, h2, dinv, bg8, w1, b1_8, w2, b2_8, bnw8, bnb8)


def _comb_body(x_ref, parts_ref, h2_ref, dinv_ref, bg_ref, y_ref, st_ref):
    i = pl.program_id(0)

    @pl.when(i == 0)
    def _():
        st_ref[...] = jnp.zeros_like(st_ref)

    gcn = dinv_ref[...] * (parts_ref[0] + parts_ref[1] + h2_ref[...])
    y = x_ref[...] + gcn + bg_ref[0:1, :]
    y_ref[...] = y
    st_ref[0:1, :] += jnp.sum(y, axis=0, keepdims=True)
    st_ref[1:2, :] += jnp.sum(y * y, axis=0, keepdims=True)


def _combine(x, parts, h2, dinv, bg8):
    return pl.pallas_call(
        _comb_body,
        grid=(N // BLK,),
        in_specs=[
            pl.BlockSpec((BLK, D), lambda i: (i, 0)),
            pl.BlockSpec((2, BLK, D), lambda i: (0, i, 0)),
            pl.BlockSpec((BLK, D), lambda i: (i, 0)),
            pl.BlockSpec((BLK, 1), lambda i: (i, 0)),
            pl.BlockSpec((8, 128), lambda i: (0, 0)),
        ],
        out_specs=[
            pl.BlockSpec((BLK, D), lambda i: (i, 0)),
            pl.BlockSpec((8, 128), lambda i: (0, 0)),
        ],
        out_shape=[
            jax.ShapeDtypeStruct((N, D), jnp.float32),
            jax.ShapeDtypeStruct((8, 128), jnp.float32),
        ],
    )(x, parts, h2, dinv, bg8)


def _ff_body(y_ref, st_ref, w1_ref, b1_ref, w2_ref, b2_ref, bnw_ref, bnb_ref,
             z_ref, st2_ref):
    i = pl.program_id(0)

    @pl.when(i == 0)
    def _():
        st2_ref[...] = jnp.zeros_like(st2_ref)

    mean = st_ref[0:1, :] / N
    var = st_ref[1:2, :] / N - mean * mean
    a = bnw_ref[0:1, :] * lax.rsqrt(var + EPS)
    cshift = bnb_ref[0:1, :] - mean * a
    h1 = y_ref[...] * a + cshift
    f1 = lax.dot_general(h1, w1_ref[...], (((1,), (1,)), ((), ())),
                         preferred_element_type=jnp.float32)
    f1 = jnp.maximum(f1 + b1_ref[0:1, :], 0.0)
    z = h1 + lax.dot_general(f1, w2_ref[...], (((1,), (1,)), ((), ())),
                             preferred_element_type=jnp.float32)
    z = z + b2_ref[0:1, :]
    z_ref[...] = z
    st2_ref[0:1, :] += jnp.sum(z, axis=0, keepdims=True)
    st2_ref[1:2, :] += jnp.sum(z * z, axis=0, keepdims=True)


def _ffn(y, st, w1, b1_8, w2, b2_8, bnw8, bnb8):
    return pl.pallas_call(
        _ff_body,
        grid=(N // BLK,),
        in_specs=[
            pl.BlockSpec((BLK, D), lambda i: (i, 0)),
            pl.BlockSpec((8, 128), lambda i: (0, 0)),
            pl.BlockSpec((DFF, D), lambda i: (0, 0)),
            pl.BlockSpec((8, DFF), lambda i: (0, 0)),
            pl.BlockSpec((D, DFF), lambda i: (0, 0)),
            pl.BlockSpec((8, 128), lambda i: (0, 0)),
            pl.BlockSpec((8, 128), lambda i: (0, 0)),
            pl.BlockSpec((8, 128), lambda i: (0, 0)),
        ],
        out_specs=[
            pl.BlockSpec((BLK, D), lambda i: (i, 0)),
            pl.BlockSpec((8, 128), lambda i: (0, 0)),
        ],
        out_shape=[
            jax.ShapeDtypeStruct((N, D), jnp.float32),
            jax.ShapeDtypeStruct((8, 128), jnp.float32),
        ],
    )(y, st, w1, b1_8, w2, b2_8, bnw8, bnb8)


def _bn2_body(z_ref, st2_ref, bnw_ref, bnb_ref, o_ref):
    mean = st2_ref[0:1, :] / N
    var = st2_ref[1:2, :] / N - mean * mean
    a = bnw_ref[0:1, :] * lax.rsqrt(var + EPS)
    cshift = bnb_ref[0:1, :] - mean * a
    o_ref[...] = z_ref[...] * a + cshift


def _bn2(z, st2, bnw8, bnb8):
    return pl.pallas_call(
        _bn2_body,
        grid=(N // BLK,),
        in_specs=[
            pl.BlockSpec((BLK, D), lambda i: (i, 0)),
            pl.BlockSpec((8, 128), lambda i: (0, 0)),
            pl.BlockSpec((8, 128), lambda i: (0, 0)),
            pl.BlockSpec((8, 128), lambda i: (0, 0)),
        ],
        out_specs=pl.BlockSpec((BLK, D), lambda i: (i, 0)),
        out_shape=jax.ShapeDtypeStruct((N, D), jnp.float32),
    )(z, st2, bnw8, bnb8)


def kernel(x, edge_index, W_gcn, b_gcn, bn_w, bn_b, W1, b1, W2, b2):
    src = edge_index[0].astype(jnp.int32)
    dst = edge_index[1].astype(jnp.int32)
    pidx = jnp.arange(EP - E, dtype=jnp.int32) % 128
    srcp = jnp.concatenate([src, pidx]).reshape(NW * KB, EB)
    dstp = jnp.concatenate([dst, N + pidx]).reshape(NW * KB, EB)

    degp = _deg_kernel(dstp).reshape(2, NP, 1)
    h2, dinv = _prep(x, W_gcn, degp)
    parts = _edge_kernel(h2, srcp, dstp)

    bg8 = jnp.broadcast_to(b_gcn.reshape(1, D), (8, D))
    bnw8 = jnp.broadcast_to(bn_w.reshape(1, D), (8, D))
    bnb8 = jnp.broadcast_to(bn_b.reshape(1, D), (8, D))
    b1_8 = jnp.broadcast_to(b1.reshape(1, DFF), (8, DFF))
    b2_8 = jnp.broadcast_to(b2.reshape(1, D), (8, D))

    return _mega(x, parts, h2, dinv, bg8, W1, b1_8, W2, b2_8, bnw8, bnb8)
